# baseline (speedup 1.0000x reference)
import functools

import jax
import jax.numpy as jnp
from jax import lax
from jax.experimental import pallas as pl
from jax.experimental.pallas import tpu as pltpu
from jax.experimental.pallas import tpu_sc as plsc

D = 3
H = 2.0


def _softplus(x):
    return jnp.maximum(x, 0.0) + jnp.log1p(jnp.exp(-jnp.abs(x)))


def _silu(x):
    h = 0.5 * x
    return h * jnp.tanh(h) + h


def _mm(a, b):
    return lax.dot_general(a, b, (((1,), (0,)), ((), ())),
                           preferred_element_type=jnp.float32)


def _mlp_T(w0, b0, w1, b1, w2, b2, x, act):
    h = act(_mm(w0, x) + b0)
    h = act(_mm(w1, h) + b1)
    return _mm(w2, h) + b2


def _node_body(st_ref, dt_ref, vt_ref, w0, b0, w1, b1, w2r, b2, out_ref):
    blk = st_ref.shape[1]
    e = jnp.float32(0.01)
    st = st_ref[...]
    vt = 1.0 / dt_ref[...]
    xs = jnp.concatenate([st, st + e, st, st - e], axis=1)
    xv = jnp.concatenate([vt, vt, vt + e, vt], axis=1)
    u = _mlp_T(w0[...], b0[...], w1[...], b1[...], w2r[...], b2[...],
               jnp.concatenate([xs, xv], axis=0), _softplus)
    u0 = u[:, 0:blk]
    usp = u[:, blk:2 * blk]
    uvp = u[:, 2 * blk:3 * blk]
    usm = u[:, 3 * blk:4 * blk]
    T = (usp - u0) / e
    P = -(uvp - u0) / e
    inv_cn = (usp - 2.0 * u0 + usm) / (T * (e * e))
    pd = P * (vt[:, 0:blk] * vt[:, 0:blk])
    inv_t = 1.0 / T
    zero = jnp.zeros_like(T)
    out_ref[...] = jnp.concatenate(
        [T, pd, inv_t, inv_cn, vt_ref[...], zero], axis=0).T


def _node_stage(ST, dT, vT, ew, n_pad, blk):
    grid = n_pad // blk
    w0, b0, w1, b1, w2r, b2 = ew
    return pl.pallas_call(
        _node_body,
        grid=(grid,),
        in_specs=[
            pl.BlockSpec((1, blk), lambda p: (0, p)),
            pl.BlockSpec((1, blk), lambda p: (0, p)),
            pl.BlockSpec((3, blk), lambda p: (0, p)),
            pl.BlockSpec((32, 2), lambda p: (0, 0)),
            pl.BlockSpec((32, 1), lambda p: (0, 0)),
            pl.BlockSpec((32, 32), lambda p: (0, 0)),
            pl.BlockSpec((32, 1), lambda p: (0, 0)),
            pl.BlockSpec((1, 32), lambda p: (0, 0)),
            pl.BlockSpec((1, 1), lambda p: (0, 0)),
        ],
        out_specs=pl.BlockSpec((blk, 8), lambda p: (p, 0)),
        out_shape=jax.ShapeDtypeStruct((n_pad, 8), jnp.float32),
    )(ST, dT, vT, w0, b0, w1, b1, w2r, b2)


def _edge_body(n_edges, blk, rt_ref, gi_ref, gj_ref,
               abw0, abb0, abw1, abb1, abw2, abb2,
               cww0, cwb0, cww1, cwb1, cww2, cwb2,
               ci_ref, cj_ref):
    p = pl.program_id(0)
    rt = rt_ref[...]
    gi = gi_ref[...].T
    gj = gj_ref[...].T
    T_i, pd_i, invT_i, invCn_i = gi[0:1], gi[1:2], gi[2:3], gi[3:4]
    T_j, pd_j, invT_j, invCn_j = gj[0:1], gj[1:2], gj[2:3], gj[3:4]
    v_ij = gi[4:7] - gj[4:7]

    rn = jnp.sqrt(jnp.sum(rt * rt, axis=0, keepdims=True))
    EPS = jnp.float32(0.001)
    s_p = jnp.abs(rn + EPS) / H
    s_m = jnp.abs(rn - EPS) / H
    xw = jnp.concatenate([s_p, s_m], axis=1)

    rh = rn / H
    xr4 = jnp.concatenate([rh, rh, rh, rh], axis=1)
    xt4 = jnp.concatenate([T_i, T_j, T_i + EPS, T_j + EPS], axis=1)
    x4 = jnp.concatenate([xr4, xt4], axis=0)
    xcw = jnp.concatenate([x4[:, 0:2 * blk], xw], axis=0)
    ab_cat = _mlp_T(abw0[...], abb0[...], abw1[...], abb1[...], abw2[...],
                    abb2[...], x4, _silu)
    cw_cat = _mlp_T(cww0[...], cwb0[...], cww1[...], cwb1[...], cww2[...],
                    cwb2[...], xcw, _silu)
    a_cat = ab_cat[0:1]
    b_cat = ab_cat[1:2]
    wout = jnp.exp(cw_cat[1:2]) * (1.0 - xw * xw)
    w_p = wout[:, 0:blk]
    w_m = wout[:, blk:2 * blk]
    dw_dr = (w_p - w_m) / (2.0 * EPS * rn)
    grad_w = dw_dr * rt
    e_ij = rt / (rn + 1e-8)
    term_pd = (pd_i + pd_j) * grad_w

    A_i = a_cat[:, 0:blk]; A_j = a_cat[:, blk:2 * blk]
    A_ie = a_cat[:, 2 * blk:3 * blk]; A_je = a_cat[:, 3 * blk:4 * blk]
    B_i = b_cat[:, 0:blk]; B_j = b_cat[:, blk:2 * blk]
    B_ie = b_cat[:, 2 * blk:3 * blk]; B_je = b_cat[:, 3 * blk:4 * blk]
    C_i = cw_cat[0:1, 0:blk]; C_j = cw_cat[0:1, blk:2 * blk]

    A_ij = A_i * A_j; B_ij = B_i * B_j; C_ij = C_i * C_j
    gA_i = 2.0 * A_ij * (A_ie * A_j - A_ij) / EPS
    gB_i = 2.0 * B_ij * (B_ie * B_j - B_ij) / EPS
    gA_j = 2.0 * A_ij * (A_i * A_je - A_ij) / EPS
    gB_j = 2.0 * B_ij * (B_i * B_je - B_ij) / EPS
    dev = jnp.sum(e_ij * v_ij, axis=0, keepdims=True)
    vv = jnp.sum(v_ij * v_ij, axis=0, keepdims=True)
    A2 = A_ij * A_ij
    BA = (B_ij * B_ij - A2) / D
    aux_v = 0.5 * A2 * v_ij + (0.5 * A2 + BA) * dev * e_ij
    invCnT_i = invCn_i * invT_i
    invCnT_j = invCn_j * invT_j
    term_msv = (invT_i + invT_j) * aux_v
    term = -(invCnT_i + invCnT_j) * aux_v
    mterm_i = (gA_i / 2 * v_ij + (gA_i / 2 + (gB_i - gA_i) / D) * dev * e_ij) * invCn_i
    mterm_j = (gA_j / 2 * v_ij + (gA_j / 2 + (gB_j - gA_j) / D) * dev * e_ij) * invCn_j
    aux_s = (A2 / 2 * vv + (A2 / 2 + BA) * (dev * dev)) / 4
    sterm_i = (gA_i / 2 * vv + (gA_i / 2 + (gB_i - gA_i) / D) * (dev * dev)) * invCn_i / 4
    sterm_j = (gA_j / 2 * vv + (gA_j / 2 + (gB_j - gA_j) / D) * (dev * dev)) * invCn_j / 4
    sterm3b = (gA_i / 2 * vv + (gA_j / 2 + (gB_i - gA_i) / D) * (dev * dev)) * invCn_i / 4

    c2 = C_ij * C_ij
    ch9_s = (invT_i + invT_j) * aux_s
    ch9_a = (invT_i - invT_j) * c2
    ch6_8 = -0.5 * (term + mterm_i + mterm_j)
    zero1 = jnp.zeros((1, term_pd.shape[1]), jnp.float32)
    ci = jnp.concatenate([
        term_pd,
        -0.5 * term_msv,
        ch6_8,
        ch9_s + ch9_a,
        -(2 * invCnT_i + invCnT_j) * aux_s + sterm_i + sterm_j,
        zero1,
    ], axis=0)
    cj = jnp.concatenate([
        -term_pd,
        0.5 * term_msv,
        -ch6_8,
        ch9_s - ch9_a,
        -(2 * invCnT_j + invCnT_i) * aux_s + sterm_j + sterm3b,
        zero1,
    ], axis=0)
    gidx = p * blk + lax.broadcasted_iota(jnp.int32, (1, blk), 1)
    mask = gidx < n_edges
    ci_ref[...] = jnp.where(mask, ci, 0.0).T
    cj_ref[...] = jnp.where(mask, cj, 0.0).T


def _edge_stage(rT, gT, weights, n_edges, e_pad, blk):
    grid = e_pad // blk
    nblk = e_pad // blk
    wspec = [
        pl.BlockSpec((64, 2), lambda p: (0, 0)),
        pl.BlockSpec((64, 1), lambda p: (0, 0)),
        pl.BlockSpec((64, 64), lambda p: (0, 0)),
        pl.BlockSpec((64, 1), lambda p: (0, 0)),
        pl.BlockSpec((2, 64), lambda p: (0, 0)),
        pl.BlockSpec((2, 1), lambda p: (0, 0)),
        pl.BlockSpec((64, 3), lambda p: (0, 0)),
        pl.BlockSpec((64, 1), lambda p: (0, 0)),
        pl.BlockSpec((64, 64), lambda p: (0, 0)),
        pl.BlockSpec((64, 1), lambda p: (0, 0)),
        pl.BlockSpec((2, 64), lambda p: (0, 0)),
        pl.BlockSpec((2, 1), lambda p: (0, 0)),
    ]
    out_sds = jax.ShapeDtypeStruct((e_pad, 12), jnp.float32)
    return pl.pallas_call(
        functools.partial(_edge_body, n_edges, blk),
        grid=(grid,),
        in_specs=[
            pl.BlockSpec((3, blk), lambda p: (0, p)),
            pl.BlockSpec((blk, 8), lambda p: (p, 0)),
            pl.BlockSpec((blk, 8), lambda p: (p + nblk, 0)),
        ] + wspec,
        out_specs=[
            pl.BlockSpec((blk, 12), lambda p: (p, 0)),
            pl.BlockSpec((blk, 12), lambda p: (p, 0)),
        ],
        out_shape=[out_sds, out_sds],
    )(rT, gT, gT, *weights)


_GB = 8


def _sc_gather(node_tab, idx2d, nch_total):
    mesh = plsc.VectorSubcoreMesh(core_axis_name="c", subcore_axis_name="s")
    rows_per_w = nch_total // 32
    nb = rows_per_w // _GB
    ng = nb // 2
    gb128 = _GB * 128

    @functools.partial(
        pl.kernel,
        out_type=jax.ShapeDtypeStruct((nch_total * 128, 8), jnp.float32),
        mesh=mesh,
        compiler_params=pltpu.CompilerParams(use_tc_tiling_on_sc=False),
        scratch_types=[
            pltpu.VMEM((_GB, 128), jnp.int32),
            pltpu.VMEM((_GB, 128), jnp.int32),
            pltpu.VMEM((gb128, 8), jnp.float32),
            pltpu.VMEM((gb128, 8), jnp.float32),
            pltpu.SemaphoreType.DMA,
            pltpu.SemaphoreType.DMA,
            pltpu.SemaphoreType.DMA,
            pltpu.SemaphoreType.DMA,
        ],
    )
    def k(tab, idx, out, ib0, ib1, rb0, rb1, li0, li1, g0, g1):
        c = lax.axis_index("c")
        s = lax.axis_index("s")
        wid = c * 16 + s
        w_base = wid * rows_per_w

        pltpu.async_copy(idx.at[pl.ds(w_base, _GB)], ib0, li0)
        pltpu.async_copy(idx.at[pl.ds(w_base + _GB, _GB)], ib1, li1)

        def phase(g, bsel, ib, rb, li, gsem):
            row = w_base + (2 * g + bsel) * _GB
            pltpu.make_async_copy(idx.at[pl.ds(row, _GB)], ib, li).wait()
            descs = [
                pltpu.async_copy(tab.at[ib.at[kk]],
                                 rb.at[pl.ds(kk * 128, 128)], gsem)
                for kk in range(_GB)
            ]
            for dsc in descs:
                dsc.wait()
            pltpu.sync_copy(rb, out.at[pl.ds(row * 128, gb128)])
            nxt = 2 * g + bsel + 2

            @pl.when(nxt < nb)
            def _():
                pltpu.async_copy(
                    idx.at[pl.ds(w_base + nxt * _GB, _GB)], ib, li)

        def body(g, carry):
            phase(g, 0, ib0, rb0, li0, g0)
            phase(g, 1, ib1, rb1, li1, g1)
            return carry

        lax.fori_loop(0, ng, body, 0)

    return k(node_tab, idx2d)


_SB = 8


def _sc_scatter(ci3, cj3, idx2d, zeros, n_acc, e_pad):
    mesh = plsc.VectorSubcoreMesh(core_axis_name="c", subcore_axis_name="s")
    ch_e = e_pad // 128
    rows_per_tile = ch_e // 32
    nb = rows_per_tile // _SB
    ng = nb // 2
    has_tail = (nb % 2) == 1
    zrows = n_acc // 16

    @functools.partial(
        pl.kernel,
        out_type=jax.ShapeDtypeStruct((2, n_acc, 12), jnp.float32),
        mesh=mesh,
        compiler_params=pltpu.CompilerParams(use_tc_tiling_on_sc=False),
        scratch_types=[
            pltpu.VMEM_SHARED((n_acc, 12), jnp.float32),
            pltpu.VMEM((_SB, 128), jnp.int32),
            pltpu.VMEM((_SB, 128), jnp.int32),
            pltpu.VMEM((_SB * 128, 12), jnp.float32),
            pltpu.VMEM((_SB * 128, 12), jnp.float32),
            pltpu.SemaphoreType.DMA,
            pltpu.SemaphoreType.DMA,
            pltpu.SemaphoreType.DMA,
            pltpu.SemaphoreType.DMA,
        ],
    )
    def k(ci, cj, idx, zz, out, acc, ib0, ib1, rb0, rb1, l0, l1, s0, s1):
        c = lax.axis_index("c")
        s = lax.axis_index("s")
        pltpu.sync_copy(zz.at[pl.ds(s * zrows, zrows)],
                        acc.at[pl.ds(s * zrows, zrows)])
        plsc.subcore_barrier()

        def run_array(carr, idx_off):
            t0 = c * (ch_e // 2) + s * rows_per_tile

            def loads(row, ib, rb, lsem):
                pltpu.async_copy(idx.at[pl.ds(idx_off + row, _SB)], ib, lsem)
                pltpu.async_copy(carr.at[pl.ds(row * 128, _SB * 128)],
                                 rb, lsem)

            def wait_loads(row, ib, rb, lsem):
                pltpu.make_async_copy(
                    idx.at[pl.ds(idx_off + row, _SB)], ib, lsem).wait()
                pltpu.make_async_copy(
                    carr.at[pl.ds(row * 128, _SB * 128)], rb, lsem).wait()

            loads(t0, ib0, rb0, l0)
            loads(t0 + _SB, ib1, rb1, l1)

            def phase(g, bsel, ib, rb, lsem, ssem):
                row = t0 + (2 * g + bsel) * _SB
                wait_loads(row, ib, rb, lsem)
                descs = [
                    pltpu.async_copy(rb.at[pl.ds(kk * 128, 128)],
                                     acc.at[ib.at[kk]], ssem, add=True)
                    for kk in range(_SB)
                ]
                for dsc in descs:
                    dsc.wait()
                nxt = 2 * g + bsel + 2

                @pl.when(nxt < nb)
                def _():
                    loads(t0 + nxt * _SB, ib, rb, lsem)

            def body(g, carry):
                phase(g, 0, ib0, rb0, l0, s0)
                phase(g, 1, ib1, rb1, l1, s1)
                return carry

            lax.fori_loop(0, ng, body, 0)
            if has_tail:
                phase(ng, 0, ib0, rb0, l0, s0)

        run_array(ci, 0)
        run_array(cj, ch_e)
        plsc.subcore_barrier()
        pltpu.sync_copy(acc.at[pl.ds(s * zrows, zrows)],
                        out.at[c, pl.ds(s * zrows, zrows)])

    return k(ci3, cj3, idx2d, zeros)


def _combine_body(p0_ref, p1_ref, out_ref):
    acc = p0_ref[...] + p1_ref[...]
    out_ref[...] = acc[0, :, 0:11]


def _combine_stage(partials, n_nodes, blk):
    grid = n_nodes // blk
    return pl.pallas_call(
        _combine_body,
        grid=(grid,),
        in_specs=[
            pl.BlockSpec((1, blk, 12), lambda p: (0, p, 0)),
            pl.BlockSpec((1, blk, 12), lambda p: (1, p, 0)),
        ],
        out_specs=pl.BlockSpec((blk, 11), lambda p: (p, 0)),
        out_shape=jax.ShapeDtypeStruct((n_nodes, 11), jnp.float32),
    )(partials, partials)


def _pack_pair(la, lb, fin_a, fin_b):
    (aw0, ab0), (aw1, ab1), (aw2, ab2) = la
    (bw0, bb0), (bw1, bb1), (bw2, bb2) = lb
    z = jnp.zeros
    if fin_b == 0:
        w0 = jnp.concatenate([aw0, bw0], axis=0)
    else:
        w0 = jnp.concatenate([
            jnp.concatenate([aw0, z((32, fin_b), jnp.float32)], axis=1),
            jnp.concatenate([z((32, fin_a), jnp.float32), bw0], axis=1),
        ], axis=0)
    b0 = jnp.concatenate([ab0, bb0]).reshape(64, 1)
    z32 = z((32, 32), jnp.float32)
    w1 = jnp.concatenate([
        jnp.concatenate([aw1, z32], axis=1),
        jnp.concatenate([z32, bw1], axis=1),
    ], axis=0)
    b1 = jnp.concatenate([ab1, bb1]).reshape(64, 1)
    z1 = z((1, 32), jnp.float32)
    w2 = jnp.concatenate([
        jnp.concatenate([aw2, z1], axis=1),
        jnp.concatenate([z1, bw2], axis=1),
    ], axis=0)
    b2 = jnp.concatenate([ab2, bb2]).reshape(2, 1)
    return (w0, b0, w1, b1, w2, b2)


def kernel(v, edge_index, r_ij, S, d, dW, dV, params):
    N = v.shape[0]
    E = edge_index.shape[1]
    BLK = 2048
    N_pad = ((N + BLK - 1) // BLK) * BLK
    E_pad = ((E + 32767) // 32768) * 32768
    N_acc = ((N + 127) // 128) * 128
    f32 = jnp.float32

    ST = jnp.pad(S.astype(f32).T, ((0, 0), (0, N_pad - N)))
    dT = jnp.pad(d.astype(f32).T, ((0, 0), (0, N_pad - N)),
                 constant_values=1.0)
    vT = jnp.pad(v.astype(f32).T, ((0, 0), (0, N_pad - N)))
    (ew0, eb0), (ew1, eb1), (ew2, eb2) = params['E']
    t_sign = jnp.array([1.0, -1.0], dtype=f32)
    ew = (jnp.abs(ew0) * t_sign, eb0.reshape(32, 1), jnp.abs(ew1),
          eb1.reshape(32, 1), jnp.abs(ew2), eb2.reshape(1, 1))
    node_tab = _node_stage(ST, dT, vT, ew, N_pad, BLK)

    pad_idx = jnp.arange(E_pad - E, dtype=jnp.int32) % N
    ii = jnp.concatenate([edge_index[0], pad_idx])
    ij = jnp.concatenate([edge_index[1], pad_idx])
    idx2d = jnp.concatenate([ii, ij]).reshape(-1, 128)
    nch_total = idx2d.shape[0]
    gT = _sc_gather(node_tab, idx2d, nch_total)

    rT = jnp.pad(r_ij.astype(f32).T, ((0, 0), (0, E_pad - E)),
                 constant_values=1.0)
    weights = _pack_pair(params['A'], params['B'], 2, 0) \
        + _pack_pair(params['C'], params['W'], 2, 1)
    ci_r, cj_r = _edge_stage(rT, gT, weights, E, E_pad, BLK)

    zeros = jnp.zeros((N_acc, 12), f32)
    partials = _sc_scatter(ci_r, cj_r, idx2d, zeros, N_acc, E_pad)

    return _combine_stage(partials, N, 2000)


# speedup vs baseline: 1.2287x; 1.2287x over previous
import functools

import jax
import jax.numpy as jnp
from jax import lax
from jax.experimental import pallas as pl
from jax.experimental.pallas import tpu as pltpu
from jax.experimental.pallas import tpu_sc as plsc

D = 3
H = 2.0
_CW = 16


def _softplus(x):
    return jnp.maximum(x, 0.0) + jnp.log1p(jnp.exp(-jnp.abs(x)))


def _silu(x):
    h = 0.5 * x
    return h * jnp.tanh(h) + h


def _mm(a, b):
    return lax.dot_general(a, b, (((1,), (0,)), ((), ())),
                           preferred_element_type=jnp.float32)


def _mlp_T(w0, b0, w1, b1, w2, b2, x, act):
    h = act(_mm(w0, x) + b0)
    h = act(_mm(w1, h) + b1)
    return _mm(w2, h) + b2


def _node_body(st_ref, dt_ref, vt_ref, w0, b0, w1, b1, w2r, b2, out_ref):
    blk = st_ref.shape[1]
    e = jnp.float32(0.01)
    st = st_ref[...]
    vt = 1.0 / dt_ref[...]
    xs = jnp.concatenate([st, st + e, st, st - e], axis=1)
    xv = jnp.concatenate([vt, vt, vt + e, vt], axis=1)
    u = _mlp_T(w0[...], b0[...], w1[...], b1[...], w2r[...], b2[...],
               jnp.concatenate([xs, xv], axis=0), _softplus)
    u0 = u[:, 0:blk]
    usp = u[:, blk:2 * blk]
    uvp = u[:, 2 * blk:3 * blk]
    usm = u[:, 3 * blk:4 * blk]
    T = (usp - u0) / e
    P = -(uvp - u0) / e
    inv_cn = (usp - 2.0 * u0 + usm) / (T * (e * e))
    pd = P * (vt[:, 0:blk] * vt[:, 0:blk])
    inv_t = 1.0 / T
    zero = jnp.zeros_like(T)
    out_ref[...] = jnp.concatenate(
        [T, pd, inv_t, inv_cn, vt_ref[...], zero], axis=0).T


def _node_stage(ST, dT, vT, ew, n_pad, blk):
    grid = n_pad // blk
    w0, b0, w1, b1, w2r, b2 = ew
    return pl.pallas_call(
        _node_body,
        grid=(grid,),
        in_specs=[
            pl.BlockSpec((1, blk), lambda p: (0, p)),
            pl.BlockSpec((1, blk), lambda p: (0, p)),
            pl.BlockSpec((3, blk), lambda p: (0, p)),
            pl.BlockSpec((32, 2), lambda p: (0, 0)),
            pl.BlockSpec((32, 1), lambda p: (0, 0)),
            pl.BlockSpec((32, 32), lambda p: (0, 0)),
            pl.BlockSpec((32, 1), lambda p: (0, 0)),
            pl.BlockSpec((1, 32), lambda p: (0, 0)),
            pl.BlockSpec((1, 1), lambda p: (0, 0)),
        ],
        out_specs=pl.BlockSpec((blk, 8), lambda p: (p, 0)),
        out_shape=jax.ShapeDtypeStruct((n_pad, 8), jnp.float32),
    )(ST, dT, vT, w0, b0, w1, b1, w2r, b2)


def _edge_body(n_edges, blk, rt_ref, gi_ref, gj_ref,
               abw0, abb0, abw1, abb1, abw2, abb2,
               cww0, cwb0, cww1, cwb1, cww2, cwb2,
               ci_ref, cj_ref):
    p = pl.program_id(0)
    rt = rt_ref[...]
    gi = gi_ref[...].T
    gj = gj_ref[...].T
    T_i, pd_i, invT_i, invCn_i = gi[0:1], gi[1:2], gi[2:3], gi[3:4]
    T_j, pd_j, invT_j, invCn_j = gj[0:1], gj[1:2], gj[2:3], gj[3:4]
    v_ij = gi[4:7] - gj[4:7]

    rn = jnp.sqrt(jnp.sum(rt * rt, axis=0, keepdims=True))
    EPS = jnp.float32(0.001)
    s_p = jnp.abs(rn + EPS) / H
    s_m = jnp.abs(rn - EPS) / H
    xw = jnp.concatenate([s_p, s_m], axis=1)

    rh = rn / H
    xr4 = jnp.concatenate([rh, rh, rh, rh], axis=1)
    xt4 = jnp.concatenate([T_i, T_j, T_i + EPS, T_j + EPS], axis=1)
    x4 = jnp.concatenate([xr4, xt4], axis=0)
    xcw = jnp.concatenate([x4[:, 0:2 * blk], xw], axis=0)
    ab_cat = _mlp_T(abw0[...], abb0[...], abw1[...], abb1[...], abw2[...],
                    abb2[...], x4, _silu)
    cw_cat = _mlp_T(cww0[...], cwb0[...], cww1[...], cwb1[...], cww2[...],
                    cwb2[...], xcw, _silu)
    a_cat = ab_cat[0:1]
    b_cat = ab_cat[1:2]
    wout = jnp.exp(cw_cat[1:2]) * (1.0 - xw * xw)
    w_p = wout[:, 0:blk]
    w_m = wout[:, blk:2 * blk]
    dw_dr = (w_p - w_m) / (2.0 * EPS * rn)
    grad_w = dw_dr * rt
    e_ij = rt / (rn + 1e-8)
    term_pd = (pd_i + pd_j) * grad_w

    A_i = a_cat[:, 0:blk]; A_j = a_cat[:, blk:2 * blk]
    A_ie = a_cat[:, 2 * blk:3 * blk]; A_je = a_cat[:, 3 * blk:4 * blk]
    B_i = b_cat[:, 0:blk]; B_j = b_cat[:, blk:2 * blk]
    B_ie = b_cat[:, 2 * blk:3 * blk]; B_je = b_cat[:, 3 * blk:4 * blk]
    C_i = cw_cat[0:1, 0:blk]; C_j = cw_cat[0:1, blk:2 * blk]

    A_ij = A_i * A_j; B_ij = B_i * B_j; C_ij = C_i * C_j
    gA_i = 2.0 * A_ij * (A_ie * A_j - A_ij) / EPS
    gB_i = 2.0 * B_ij * (B_ie * B_j - B_ij) / EPS
    gA_j = 2.0 * A_ij * (A_i * A_je - A_ij) / EPS
    gB_j = 2.0 * B_ij * (B_i * B_je - B_ij) / EPS
    dev = jnp.sum(e_ij * v_ij, axis=0, keepdims=True)
    vv = jnp.sum(v_ij * v_ij, axis=0, keepdims=True)
    A2 = A_ij * A_ij
    BA = (B_ij * B_ij - A2) / D
    aux_v = 0.5 * A2 * v_ij + (0.5 * A2 + BA) * dev * e_ij
    invCnT_i = invCn_i * invT_i
    invCnT_j = invCn_j * invT_j
    term_msv = (invT_i + invT_j) * aux_v
    term = -(invCnT_i + invCnT_j) * aux_v
    mterm_i = (gA_i / 2 * v_ij + (gA_i / 2 + (gB_i - gA_i) / D) * dev * e_ij) * invCn_i
    mterm_j = (gA_j / 2 * v_ij + (gA_j / 2 + (gB_j - gA_j) / D) * dev * e_ij) * invCn_j
    aux_s = (A2 / 2 * vv + (A2 / 2 + BA) * (dev * dev)) / 4
    sterm_i = (gA_i / 2 * vv + (gA_i / 2 + (gB_i - gA_i) / D) * (dev * dev)) * invCn_i / 4
    sterm_j = (gA_j / 2 * vv + (gA_j / 2 + (gB_j - gA_j) / D) * (dev * dev)) * invCn_j / 4
    sterm3b = (gA_i / 2 * vv + (gA_j / 2 + (gB_i - gA_i) / D) * (dev * dev)) * invCn_i / 4

    c2 = C_ij * C_ij
    ch9_s = (invT_i + invT_j) * aux_s
    ch9_a = (invT_i - invT_j) * c2
    ch6_8 = -0.5 * (term + mterm_i + mterm_j)
    zpad = jnp.zeros((_CW - 11, term_pd.shape[1]), jnp.float32)
    ci = jnp.concatenate([
        term_pd,
        -0.5 * term_msv,
        ch6_8,
        ch9_s + ch9_a,
        -(2 * invCnT_i + invCnT_j) * aux_s + sterm_i + sterm_j,
        zpad,
    ], axis=0)
    cj = jnp.concatenate([
        -term_pd,
        0.5 * term_msv,
        -ch6_8,
        ch9_s - ch9_a,
        -(2 * invCnT_j + invCnT_i) * aux_s + sterm_j + sterm3b,
        zpad,
    ], axis=0)
    gidx = p * blk + lax.broadcasted_iota(jnp.int32, (1, blk), 1)
    mask = gidx < n_edges
    ci_ref[...] = jnp.where(mask, ci, 0.0).T
    cj_ref[...] = jnp.where(mask, cj, 0.0).T


def _edge_stage(rT, gT, weights, n_edges, e_pad, blk):
    grid = e_pad // blk
    nblk = e_pad // blk
    wspec = [
        pl.BlockSpec((64, 2), lambda p: (0, 0)),
        pl.BlockSpec((64, 1), lambda p: (0, 0)),
        pl.BlockSpec((64, 64), lambda p: (0, 0)),
        pl.BlockSpec((64, 1), lambda p: (0, 0)),
        pl.BlockSpec((2, 64), lambda p: (0, 0)),
        pl.BlockSpec((2, 1), lambda p: (0, 0)),
        pl.BlockSpec((64, 3), lambda p: (0, 0)),
        pl.BlockSpec((64, 1), lambda p: (0, 0)),
        pl.BlockSpec((64, 64), lambda p: (0, 0)),
        pl.BlockSpec((64, 1), lambda p: (0, 0)),
        pl.BlockSpec((2, 64), lambda p: (0, 0)),
        pl.BlockSpec((2, 1), lambda p: (0, 0)),
    ]
    out_sds = jax.ShapeDtypeStruct((e_pad, _CW), jnp.float32)
    return pl.pallas_call(
        functools.partial(_edge_body, n_edges, blk),
        grid=(grid,),
        in_specs=[
            pl.BlockSpec((3, blk), lambda p: (0, p)),
            pl.BlockSpec((blk, 8), lambda p: (p, 0)),
            pl.BlockSpec((blk, 8), lambda p: (p + nblk, 0)),
        ] + wspec,
        out_specs=[
            pl.BlockSpec((blk, _CW), lambda p: (p, 0)),
            pl.BlockSpec((blk, _CW), lambda p: (p, 0)),
        ],
        out_shape=[out_sds, out_sds],
    )(rT, gT, gT, *weights)


_GB = 8


def _sc_gather(node_tab, idx2d, nch_total):
    mesh = plsc.VectorSubcoreMesh(core_axis_name="c", subcore_axis_name="s")
    rows_per_w = nch_total // 32
    nb = rows_per_w // _GB
    ng = nb // 2
    gb128 = _GB * 128

    @functools.partial(
        pl.kernel,
        out_type=jax.ShapeDtypeStruct((nch_total * 128, 8), jnp.float32),
        mesh=mesh,
        compiler_params=pltpu.CompilerParams(use_tc_tiling_on_sc=False),
        scratch_types=[
            pltpu.VMEM((_GB, 128), jnp.int32),
            pltpu.VMEM((_GB, 128), jnp.int32),
            pltpu.VMEM((gb128, 8), jnp.float32),
            pltpu.VMEM((gb128, 8), jnp.float32),
            pltpu.SemaphoreType.DMA,
            pltpu.SemaphoreType.DMA,
            pltpu.SemaphoreType.DMA,
            pltpu.SemaphoreType.DMA,
        ],
    )
    def k(tab, idx, out, ib0, ib1, rb0, rb1, li0, li1, g0, g1):
        c = lax.axis_index("c")
        s = lax.axis_index("s")
        wid = c * 16 + s
        w_base = wid * rows_per_w

        pltpu.async_copy(idx.at[pl.ds(w_base, _GB)], ib0, li0)
        pltpu.async_copy(idx.at[pl.ds(w_base + _GB, _GB)], ib1, li1)

        def phase(g, bsel, ib, rb, li, gsem):
            row = w_base + (2 * g + bsel) * _GB
            pltpu.make_async_copy(idx.at[pl.ds(row, _GB)], ib, li).wait()
            descs = [
                pltpu.async_copy(tab.at[ib.at[kk]],
                                 rb.at[pl.ds(kk * 128, 128)], gsem)
                for kk in range(_GB)
            ]
            for dsc in descs:
                dsc.wait()
            pltpu.sync_copy(rb, out.at[pl.ds(row * 128, gb128)])
            nxt = 2 * g + bsel + 2

            @pl.when(nxt < nb)
            def _():
                pltpu.async_copy(
                    idx.at[pl.ds(w_base + nxt * _GB, _GB)], ib, li)

        def body(g, carry):
            phase(g, 0, ib0, rb0, li0, g0)
            phase(g, 1, ib1, rb1, li1, g1)
            return carry

        lax.fori_loop(0, ng, body, 0)

    return k(node_tab, idx2d)


_SB = 8


def _sc_scatter(ci3, cj3, idx2d, zeros, n_acc, e_pad):
    mesh = plsc.VectorSubcoreMesh(core_axis_name="c", subcore_axis_name="s")
    ch_e = e_pad // 128
    rows_per_tile = ch_e // 32
    nb = rows_per_tile // _SB
    ng = nb // 2
    has_tail = (nb % 2) == 1
    zrows = n_acc // 16

    @functools.partial(
        pl.kernel,
        out_type=jax.ShapeDtypeStruct((2, n_acc, _CW), jnp.float32),
        mesh=mesh,
        compiler_params=pltpu.CompilerParams(use_tc_tiling_on_sc=False),
        scratch_types=[
            pltpu.VMEM_SHARED((n_acc, _CW), jnp.float32),
            pltpu.VMEM((_SB, 128), jnp.int32),
            pltpu.VMEM((_SB, 128), jnp.int32),
            pltpu.VMEM((_SB * 128, _CW), jnp.float32),
            pltpu.VMEM((_SB * 128, _CW), jnp.float32),
            pltpu.SemaphoreType.DMA,
            pltpu.SemaphoreType.DMA,
            pltpu.SemaphoreType.DMA,
            pltpu.SemaphoreType.DMA,
        ],
    )
    def k(ci, cj, idx, zz, out, acc, ib0, ib1, rb0, rb1, l0, l1, s0, s1):
        c = lax.axis_index("c")
        s = lax.axis_index("s")
        pltpu.sync_copy(zz.at[pl.ds(s * zrows, zrows)],
                        acc.at[pl.ds(s * zrows, zrows)])
        plsc.subcore_barrier()

        def run_array(carr, idx_off):
            t0 = c * (ch_e // 2) + s * rows_per_tile

            def loads(row, ib, rb, lsem):
                pltpu.async_copy(idx.at[pl.ds(idx_off + row, _SB)], ib, lsem)
                pltpu.async_copy(carr.at[pl.ds(row * 128, _SB * 128)],
                                 rb, lsem)

            def wait_loads(row, ib, rb, lsem):
                pltpu.make_async_copy(
                    idx.at[pl.ds(idx_off + row, _SB)], ib, lsem).wait()
                pltpu.make_async_copy(
                    carr.at[pl.ds(row * 128, _SB * 128)], rb, lsem).wait()

            loads(t0, ib0, rb0, l0)
            loads(t0 + _SB, ib1, rb1, l1)

            def phase(g, bsel, ib, rb, lsem, ssem):
                row = t0 + (2 * g + bsel) * _SB
                wait_loads(row, ib, rb, lsem)
                descs = [
                    pltpu.async_copy(rb.at[pl.ds(kk * 128, 128)],
                                     acc.at[ib.at[kk]], ssem, add=True)
                    for kk in range(_SB)
                ]
                for dsc in descs:
                    dsc.wait()
                nxt = 2 * g + bsel + 2

                @pl.when(nxt < nb)
                def _():
                    loads(t0 + nxt * _SB, ib, rb, lsem)

            def body(g, carry):
                phase(g, 0, ib0, rb0, l0, s0)
                phase(g, 1, ib1, rb1, l1, s1)
                return carry

            lax.fori_loop(0, ng, body, 0)
            if has_tail:
                phase(ng, 0, ib0, rb0, l0, s0)

        run_array(ci, 0)
        run_array(cj, ch_e)
        plsc.subcore_barrier()
        pltpu.sync_copy(acc.at[pl.ds(s * zrows, zrows)],
                        out.at[c, pl.ds(s * zrows, zrows)])

    return k(ci3, cj3, idx2d, zeros)


def _combine_body(*refs):
    out_ref = refs[-1]
    acc = refs[0][...]
    for r in refs[1:-1]:
        acc = acc + r[...]
    out_ref[...] = acc[0, :, 0:11]


def _combine_stage(parts, n_nodes, blk):
    grid = n_nodes // blk
    in_specs, args = [], []
    for prt in parts:
        in_specs += [pl.BlockSpec((1, blk, _CW), lambda p: (0, p, 0)),
                     pl.BlockSpec((1, blk, _CW), lambda p: (1, p, 0))]
        args += [prt, prt]
    return pl.pallas_call(
        _combine_body,
        grid=(grid,),
        in_specs=in_specs,
        out_specs=pl.BlockSpec((blk, 11), lambda p: (p, 0)),
        out_shape=jax.ShapeDtypeStruct((n_nodes, 11), jnp.float32),
    )(*args)


def _pack_pair(la, lb, fin_a, fin_b):
    (aw0, ab0), (aw1, ab1), (aw2, ab2) = la
    (bw0, bb0), (bw1, bb1), (bw2, bb2) = lb
    z = jnp.zeros
    if fin_b == 0:
        w0 = jnp.concatenate([aw0, bw0], axis=0)
    else:
        w0 = jnp.concatenate([
            jnp.concatenate([aw0, z((32, fin_b), jnp.float32)], axis=1),
            jnp.concatenate([z((32, fin_a), jnp.float32), bw0], axis=1),
        ], axis=0)
    b0 = jnp.concatenate([ab0, bb0]).reshape(64, 1)
    z32 = z((32, 32), jnp.float32)
    w1 = jnp.concatenate([
        jnp.concatenate([aw1, z32], axis=1),
        jnp.concatenate([z32, bw1], axis=1),
    ], axis=0)
    b1 = jnp.concatenate([ab1, bb1]).reshape(64, 1)
    z1 = z((1, 32), jnp.float32)
    w2 = jnp.concatenate([
        jnp.concatenate([aw2, z1], axis=1),
        jnp.concatenate([z1, bw2], axis=1),
    ], axis=0)
    b2 = jnp.concatenate([ab2, bb2]).reshape(2, 1)
    return (w0, b0, w1, b1, w2, b2)


def kernel(v, edge_index, r_ij, S, d, dW, dV, params):
    N = v.shape[0]
    E = edge_index.shape[1]
    BLK = 2048
    N_pad = ((N + BLK - 1) // BLK) * BLK
    E_pad = ((E + 32767) // 32768) * 32768
    N_acc = ((N + 127) // 128) * 128
    f32 = jnp.float32

    ST = jnp.pad(S.astype(f32).T, ((0, 0), (0, N_pad - N)))
    dT = jnp.pad(d.astype(f32).T, ((0, 0), (0, N_pad - N)),
                 constant_values=1.0)
    vT = jnp.pad(v.astype(f32).T, ((0, 0), (0, N_pad - N)))
    (ew0, eb0), (ew1, eb1), (ew2, eb2) = params['E']
    t_sign = jnp.array([1.0, -1.0], dtype=f32)
    ew = (jnp.abs(ew0) * t_sign, eb0.reshape(32, 1), jnp.abs(ew1),
          eb1.reshape(32, 1), jnp.abs(ew2), eb2.reshape(1, 1))
    node_tab = _node_stage(ST, dT, vT, ew, N_pad, BLK)

    pad_idx = jnp.arange(E_pad - E, dtype=jnp.int32) % N
    ii = jnp.concatenate([edge_index[0], pad_idx])
    ij = jnp.concatenate([edge_index[1], pad_idx])
    rT = jnp.pad(r_ij.astype(f32).T, ((0, 0), (0, E_pad - E)),
                 constant_values=1.0)
    weights = _pack_pair(params['A'], params['B'], 2, 0) \
        + _pack_pair(params['C'], params['W'], 2, 1)
    zeros = jnp.zeros((N_acc, _CW), f32)

    units = E_pad // 32768
    H1 = ((units + 1) // 2) * 32768
    halves = [(0, H1)]
    if E_pad > H1:
        halves.append((H1, E_pad - H1))
    partial_list = []
    for off, hlen in halves:
        idx2d_h = jnp.concatenate(
            [lax.slice(ii, (off,), (off + hlen,)),
             lax.slice(ij, (off,), (off + hlen,))]).reshape(-1, 128)
        g_h = _sc_gather(node_tab, idx2d_h, idx2d_h.shape[0])
        rT_h = lax.slice(rT, (0, off), (3, off + hlen))
        n_valid = min(max(E - off, 0), hlen)
        ci_h, cj_h = _edge_stage(rT_h, g_h, weights, n_valid, hlen, BLK)
        partial_list.append(
            _sc_scatter(ci_h, cj_h, idx2d_h, zeros, N_acc, hlen))

    return _combine_stage(partial_list, N, 2000)
